# baseline (speedup 1.0000x reference)
import functools

import jax
import jax.numpy as jnp
from jax import lax
from jax.experimental import pallas as pl
from jax.experimental.pallas import tpu as pltpu
from jax.experimental.pallas import tpu_sc as plsc

NUM_EMB = 100000
DIM = 64
E = 16384

NUM_CORES = 2
NUM_SUBCORES = 16
LANES = 16
FPC = DIM // NUM_CORES
FPS = FPC // NUM_SUBCORES
AROWS = E // 128
CHUNK = 4096
NCH = E // CHUNK
CROWS = CHUNK // 128
SROWS = AROWS // (NUM_CORES * NUM_SUBCORES)


def _sc_body(eidx_hbm, tabT_hbm, out_hbm, part_hbm, flag_hbm,
             rowbuf, idx_ch, acc_v, idxid_v, red_v, tmp_v, fbuf_v,
             idx_sh, spsum):
    core = lax.axis_index("c")
    sub = lax.axis_index("s")

    lanes = lax.iota(jnp.int32, LANES)

    @pl.when(sub == 0)
    def _():
        izero = jnp.zeros((LANES,), jnp.int32)
        for j in range(128 // LANES):
            fbuf_v[pl.ds(j * LANES, LANES)] = izero
        pltpu.sync_copy(fbuf_v, flag_hbm.at[core])

    side = sub // 8
    k = sub % 8
    pltpu.sync_copy(eidx_hbm.at[pl.ds(side * E + k * 2048, 2048)],
                    idx_sh.at[side, pl.ds(k * 2048, 2048)])

    for j in range(AROWS // LANES):
        idxid_v[pl.ds(j * LANES, LANES)] = j * LANES + lanes

    zero = jnp.zeros((LANES,), jnp.float32)
    for r in range(2 * SROWS):
        for kk in range(128 // LANES):
            red_v[r, pl.ds(kk * LANES, LANES)] = zero
    pltpu.sync_copy(red_v, spsum.at[pl.ds(sub * 2 * SROWS, 2 * SROWS)])

    plsc.subcore_barrier()

    for cl in range(FPS):
        feat = core * FPC + sub * FPS + cl
        pltpu.sync_copy(tabT_hbm.at[feat], rowbuf)

        for ch in range(NCH):
            pltpu.sync_copy(idx_sh.at[0, pl.ds(ch * CHUNK, CHUNK)],
                            idx_ch.at[0])
            pltpu.sync_copy(idx_sh.at[1, pl.ds(ch * CHUNK, CHUNK)],
                            idx_ch.at[1])

            @pl.loop(0, CROWS)
            def _(r):
                base = r * 128
                for kk in range(128 // LANES):
                    off = base + kk * LANES
                    ia = idx_ch[0, pl.ds(off, LANES)]
                    ib = idx_ch[1, pl.ds(off, LANES)]
                    p = (plsc.load_gather(rowbuf, [ia])
                         * plsc.load_gather(rowbuf, [ib]))
                    dst = pl.ds(kk * LANES, LANES)
                    row = ch * CROWS + r
                    if cl == 0:
                        acc_v[row, dst] = p
                    else:
                        acc_v[row, dst] = acc_v[row, dst] + p

    pltpu.sync_copy(acc_v, spsum.at[idxid_v], add=True)
    plsc.subcore_barrier()

    pltpu.sync_copy(spsum.at[pl.ds(sub * (AROWS // NUM_SUBCORES),
                                   AROWS // NUM_SUBCORES)],
                    part_hbm.at[core, pl.ds(sub * (AROWS // NUM_SUBCORES),
                                            AROWS // NUM_SUBCORES)])
    plsc.subcore_barrier()

    @pl.when(sub == 0)
    def _():
        ione = jnp.ones((LANES,), jnp.int32)
        for j in range(128 // LANES):
            fbuf_v[pl.ds(j * LANES, LANES)] = ione
        pltpu.sync_copy(fbuf_v, flag_hbm.at[core])

        def poll_cond(seen):
            return seen == 0

        def poll_body(seen):
            pltpu.sync_copy(flag_hbm.at[1 - core], fbuf_v)
            return fbuf_v[pl.ds(0, LANES)][0]

        lax.while_loop(poll_cond, poll_body, jnp.int32(0))

    plsc.subcore_barrier()

    row0 = core * NUM_SUBCORES * SROWS + sub * SROWS
    pltpu.sync_copy(spsum.at[pl.ds(row0, SROWS)], red_v.at[pl.ds(0, SROWS)])
    pltpu.sync_copy(part_hbm.at[1 - core, pl.ds(row0, SROWS)], tmp_v)
    for r in range(SROWS):
        for kk in range(128 // LANES):
            s = pl.ds(kk * LANES, LANES)
            d = red_v[r, s] + tmp_v[r, s]
            red_v[r, s] = 1.0 / (1.0 + jnp.exp(-d))
    pltpu.sync_copy(red_v.at[pl.ds(0, SROWS)], out_hbm.at[core, sub])


def kernel(edges, emb_table):
    eidx = edges.astype(jnp.int32).reshape(2 * E)
    tabT = emb_table.T
    mesh = plsc.VectorSubcoreMesh(core_axis_name="c", subcore_axis_name="s")
    sc = functools.partial(
        pl.kernel,
        mesh=mesh,
        compiler_params=pltpu.CompilerParams(needs_layout_passes=False),
        out_type=(
            jax.ShapeDtypeStruct(
                (NUM_CORES, NUM_SUBCORES, SROWS, 128), jnp.float32),
            jax.ShapeDtypeStruct((NUM_CORES, AROWS, 128), jnp.float32),
            jax.ShapeDtypeStruct((NUM_CORES, 128), jnp.int32),
        ),
        scratch_types=[
            pltpu.VMEM((NUM_EMB,), jnp.float32),
            pltpu.VMEM((2, CHUNK), jnp.int32),
            pltpu.VMEM((AROWS, 128), jnp.float32),
            pltpu.VMEM((AROWS,), jnp.int32),
            pltpu.VMEM((2 * SROWS, 128), jnp.float32),
            pltpu.VMEM((SROWS, 128), jnp.float32),
            pltpu.VMEM((128,), jnp.int32),
            pltpu.VMEM_SHARED((2, E), jnp.int32),
            pltpu.VMEM_SHARED((AROWS, 128), jnp.float32),
        ],
    )(_sc_body)
    out, _, _ = sc(eidx, tabT)
    return out.reshape(E)


# speedup vs baseline: 1.0230x; 1.0230x over previous
import functools

import jax
import jax.numpy as jnp
from jax import lax
from jax.experimental import pallas as pl
from jax.experimental.pallas import tpu as pltpu
from jax.experimental.pallas import tpu_sc as plsc

NUM_EMB = 100000
DIM = 64
E = 16384

NUM_CORES = 2
NUM_SUBCORES = 16
LANES = 16
FPC = DIM // NUM_CORES
FPS = FPC // NUM_SUBCORES
AROWS = E // 128
CHUNK = 4096
NCH = E // CHUNK
CROWS = CHUNK // 128
SROWS = AROWS // (NUM_CORES * NUM_SUBCORES)
NSTEP = FPS * NCH


def _sc_body(eidx_hbm, tabT_hbm, out_hbm, part_hbm, flag_hbm,
             rowbuf, idx_ch, acc_v, idxid_v, red_v, tmp_v, fbuf_v,
             sa0, sb0, sa1, sb1, idx_sh, spsum):
    core = lax.axis_index("c")
    sub = lax.axis_index("s")

    lanes = lax.iota(jnp.int32, LANES)

    @pl.when(sub == 0)
    def _():
        izero = jnp.zeros((LANES,), jnp.int32)
        for j in range(128 // LANES):
            fbuf_v[pl.ds(j * LANES, LANES)] = izero
        pltpu.sync_copy(fbuf_v, flag_hbm.at[core])

    side = sub // 8
    k = sub % 8
    pltpu.sync_copy(eidx_hbm.at[pl.ds(side * E + k * 2048, 2048)],
                    idx_sh.at[side, pl.ds(k * 2048, 2048)])

    for r in range(NCH):
        for j in range(CROWS // LANES):
            idxid_v[r, pl.ds(j * LANES, LANES)] = (
                r * CROWS + j * LANES + lanes)

    zero = jnp.zeros((LANES,), jnp.float32)
    for r in range(2 * SROWS):
        for kk in range(128 // LANES):
            red_v[r, pl.ds(kk * LANES, LANES)] = zero
    pltpu.sync_copy(red_v, spsum.at[pl.ds(sub * 2 * SROWS, 2 * SROWS)])

    plsc.subcore_barrier()

    sems = [(sa0, sb0), (sa1, sb1)]

    def fire(i):
        slot = i % 2
        ch = i % NCH
        ca = pltpu.async_copy(idx_sh.at[0, pl.ds(ch * CHUNK, CHUNK)],
                              idx_ch.at[slot, 0], sems[slot][0])
        cb = pltpu.async_copy(idx_sh.at[1, pl.ds(ch * CHUNK, CHUNK)],
                              idx_ch.at[slot, 1], sems[slot][1])
        return ca, cb

    pend = {0: fire(0), 1: fire(1)}
    for i in range(NSTEP):
        cl, ch = divmod(i, NCH)
        if ch == 0:
            feat = core * FPC + sub * FPS + cl
            pltpu.sync_copy(tabT_hbm.at[feat], rowbuf)
        slot = i % 2
        ca, cb = pend.pop(i)
        ca.wait()
        cb.wait()

        @pl.loop(0, CROWS)
        def _(r):
            base = r * 128
            for kk in range(128 // LANES):
                off = base + kk * LANES
                ia = idx_ch[slot, 0, pl.ds(off, LANES)]
                ib = idx_ch[slot, 1, pl.ds(off, LANES)]
                p = (plsc.load_gather(rowbuf, [ia])
                     * plsc.load_gather(rowbuf, [ib]))
                acc_v[r, pl.ds(kk * LANES, LANES)] = p

        pltpu.sync_copy(acc_v, spsum.at[idxid_v.at[ch]], add=True)
        if i + 2 < NSTEP:
            pend[i + 2] = fire(i + 2)

    plsc.subcore_barrier()

    prows = AROWS // NUM_SUBCORES
    pltpu.sync_copy(spsum.at[pl.ds(sub * prows, prows)],
                    part_hbm.at[core, pl.ds(sub * prows, prows)])
    plsc.subcore_barrier()

    @pl.when(sub == 0)
    def _():
        ione = jnp.ones((LANES,), jnp.int32)
        for j in range(128 // LANES):
            fbuf_v[pl.ds(j * LANES, LANES)] = ione
        pltpu.sync_copy(fbuf_v, flag_hbm.at[core])

        def poll_cond(seen):
            return seen == 0

        def poll_body(seen):
            pltpu.sync_copy(flag_hbm.at[1 - core], fbuf_v)
            return fbuf_v[pl.ds(0, LANES)][0]

        lax.while_loop(poll_cond, poll_body, jnp.int32(0))

    plsc.subcore_barrier()

    row0 = core * NUM_SUBCORES * SROWS + sub * SROWS
    pltpu.sync_copy(spsum.at[pl.ds(row0, SROWS)], red_v.at[pl.ds(0, SROWS)])
    pltpu.sync_copy(part_hbm.at[1 - core, pl.ds(row0, SROWS)], tmp_v)
    for r in range(SROWS):
        for kk in range(128 // LANES):
            s = pl.ds(kk * LANES, LANES)
            d = red_v[r, s] + tmp_v[r, s]
            red_v[r, s] = 1.0 / (1.0 + jnp.exp(-d))
    pltpu.sync_copy(red_v.at[pl.ds(0, SROWS)], out_hbm.at[core, sub])


def kernel(edges, emb_table):
    eidx = edges.astype(jnp.int32).reshape(2 * E)
    tabT = emb_table.T
    mesh = plsc.VectorSubcoreMesh(core_axis_name="c", subcore_axis_name="s")
    sc = functools.partial(
        pl.kernel,
        mesh=mesh,
        compiler_params=pltpu.CompilerParams(needs_layout_passes=False),
        out_type=(
            jax.ShapeDtypeStruct(
                (NUM_CORES, NUM_SUBCORES, SROWS, 128), jnp.float32),
            jax.ShapeDtypeStruct((NUM_CORES, AROWS, 128), jnp.float32),
            jax.ShapeDtypeStruct((NUM_CORES, 128), jnp.int32),
        ),
        scratch_types=[
            pltpu.VMEM((NUM_EMB,), jnp.float32),
            pltpu.VMEM((2, 2, CHUNK), jnp.int32),
            pltpu.VMEM((CROWS, 128), jnp.float32),
            pltpu.VMEM((NCH, CROWS), jnp.int32),
            pltpu.VMEM((2 * SROWS, 128), jnp.float32),
            pltpu.VMEM((SROWS, 128), jnp.float32),
            pltpu.VMEM((128,), jnp.int32),
            pltpu.SemaphoreType.DMA,
            pltpu.SemaphoreType.DMA,
            pltpu.SemaphoreType.DMA,
            pltpu.SemaphoreType.DMA,
            pltpu.VMEM_SHARED((2, E), jnp.int32),
            pltpu.VMEM_SHARED((AROWS, 128), jnp.float32),
        ],
    )(_sc_body)
    out, _, _ = sc(eidx, tabT)
    return out.reshape(E)


# speedup vs baseline: 1.0558x; 1.0321x over previous
import functools

import jax
import jax.numpy as jnp
from jax import lax
from jax.experimental import pallas as pl
from jax.experimental.pallas import tpu as pltpu
from jax.experimental.pallas import tpu_sc as plsc

NUM_EMB = 100000
DIM = 64
E = 16384

NUM_CORES = 2
NUM_SUBCORES = 16
LANES = 16
FPC = DIM // NUM_CORES
FPS = FPC // NUM_SUBCORES
AROWS = E // 128
CHUNK = 4096
NCH = E // CHUNK
CROWS = CHUNK // 128
SROWS = AROWS // (NUM_CORES * NUM_SUBCORES)
NSTEP = FPS * NCH


def _sc_body(eidx_hbm, tabT_hbm, out_hbm, part_hbm, flag_hbm,
             rowbuf, idx_ch, acc_v, idxid_v, red_v, tmp_v, fbuf_v,
             sa0, sb0, sa1, sb1, sadd0, sadd1, idx_sh, spsum):
    core = lax.axis_index("c")
    sub = lax.axis_index("s")

    lanes = lax.iota(jnp.int32, LANES)

    @pl.when(sub == 0)
    def _():
        izero = jnp.zeros((LANES,), jnp.int32)
        for j in range(128 // LANES):
            fbuf_v[pl.ds(j * LANES, LANES)] = izero
        pltpu.sync_copy(fbuf_v, flag_hbm.at[core])

    side = sub // 8
    k = sub % 8
    pltpu.sync_copy(eidx_hbm.at[pl.ds(side * E + k * 2048, 2048)],
                    idx_sh.at[side, pl.ds(k * 2048, 2048)])

    for r in range(NCH):
        for j in range(CROWS // LANES):
            idxid_v[r, pl.ds(j * LANES, LANES)] = (
                r * CROWS + j * LANES + lanes)

    zero = jnp.zeros((LANES,), jnp.float32)
    for r in range(2 * SROWS):
        for kk in range(128 // LANES):
            red_v[r, pl.ds(kk * LANES, LANES)] = zero
    pltpu.sync_copy(red_v, spsum.at[pl.ds(sub * 2 * SROWS, 2 * SROWS)])

    plsc.subcore_barrier()

    sems = [(sa0, sb0), (sa1, sb1)]

    def fire(i):
        slot = i % 2
        ch = i % NCH
        ca = pltpu.async_copy(idx_sh.at[0, pl.ds(ch * CHUNK, CHUNK)],
                              idx_ch.at[slot, 0], sems[slot][0])
        cb = pltpu.async_copy(idx_sh.at[1, pl.ds(ch * CHUNK, CHUNK)],
                              idx_ch.at[slot, 1], sems[slot][1])
        return ca, cb

    addsems = [sadd0, sadd1]
    pend = {0: fire(0), 1: fire(1)}
    pend_add = {}
    for i in range(NSTEP):
        cl, ch = divmod(i, NCH)
        if ch == 0:
            feat = core * FPC + sub * FPS + cl
            pltpu.sync_copy(tabT_hbm.at[feat], rowbuf)
        slot = i % 2
        ca, cb = pend.pop(i)
        ca.wait()
        cb.wait()
        if i - 2 in pend_add:
            pend_add.pop(i - 2).wait()

        @pl.loop(0, CROWS)
        def _(r):
            base = r * 128
            for kk in range(128 // LANES):
                off = base + kk * LANES
                ia = idx_ch[slot, 0, pl.ds(off, LANES)]
                ib = idx_ch[slot, 1, pl.ds(off, LANES)]
                p = (plsc.load_gather(rowbuf, [ia])
                     * plsc.load_gather(rowbuf, [ib]))
                acc_v[slot, r, pl.ds(kk * LANES, LANES)] = p

        pend_add[i] = pltpu.async_copy(
            acc_v.at[slot], spsum.at[idxid_v.at[ch]], addsems[slot], add=True)
        if i + 2 < NSTEP:
            pend[i + 2] = fire(i + 2)

    for c in pend_add.values():
        c.wait()
    plsc.subcore_barrier()

    prows = AROWS // NUM_SUBCORES
    pltpu.sync_copy(spsum.at[pl.ds(sub * prows, prows)],
                    part_hbm.at[core, pl.ds(sub * prows, prows)])
    plsc.subcore_barrier()

    @pl.when(sub == 0)
    def _():
        ione = jnp.ones((LANES,), jnp.int32)
        for j in range(128 // LANES):
            fbuf_v[pl.ds(j * LANES, LANES)] = ione
        pltpu.sync_copy(fbuf_v, flag_hbm.at[core])

        def poll_cond(seen):
            return seen == 0

        def poll_body(seen):
            pltpu.sync_copy(flag_hbm.at[1 - core], fbuf_v)
            return fbuf_v[pl.ds(0, LANES)][0]

        lax.while_loop(poll_cond, poll_body, jnp.int32(0))

    plsc.subcore_barrier()

    row0 = core * NUM_SUBCORES * SROWS + sub * SROWS
    pltpu.sync_copy(spsum.at[pl.ds(row0, SROWS)], red_v.at[pl.ds(0, SROWS)])
    pltpu.sync_copy(part_hbm.at[1 - core, pl.ds(row0, SROWS)], tmp_v)
    for r in range(SROWS):
        for kk in range(128 // LANES):
            s = pl.ds(kk * LANES, LANES)
            d = red_v[r, s] + tmp_v[r, s]
            red_v[r, s] = 1.0 / (1.0 + jnp.exp(-d))
    pltpu.sync_copy(red_v.at[pl.ds(0, SROWS)], out_hbm.at[core, sub])


def kernel(edges, emb_table):
    eidx = edges.astype(jnp.int32).reshape(2 * E)
    tabT = emb_table.T
    mesh = plsc.VectorSubcoreMesh(core_axis_name="c", subcore_axis_name="s")
    sc = functools.partial(
        pl.kernel,
        mesh=mesh,
        compiler_params=pltpu.CompilerParams(needs_layout_passes=False),
        out_type=(
            jax.ShapeDtypeStruct(
                (NUM_CORES, NUM_SUBCORES, SROWS, 128), jnp.float32),
            jax.ShapeDtypeStruct((NUM_CORES, AROWS, 128), jnp.float32),
            jax.ShapeDtypeStruct((NUM_CORES, 128), jnp.int32),
        ),
        scratch_types=[
            pltpu.VMEM((NUM_EMB,), jnp.float32),
            pltpu.VMEM((2, 2, CHUNK), jnp.int32),
            pltpu.VMEM((2, CROWS, 128), jnp.float32),
            pltpu.VMEM((NCH, CROWS), jnp.int32),
            pltpu.VMEM((2 * SROWS, 128), jnp.float32),
            pltpu.VMEM((SROWS, 128), jnp.float32),
            pltpu.VMEM((128,), jnp.int32),
            pltpu.SemaphoreType.DMA,
            pltpu.SemaphoreType.DMA,
            pltpu.SemaphoreType.DMA,
            pltpu.SemaphoreType.DMA,
            pltpu.SemaphoreType.DMA,
            pltpu.SemaphoreType.DMA,
            pltpu.VMEM_SHARED((2, E), jnp.int32),
            pltpu.VMEM_SHARED((AROWS, 128), jnp.float32),
        ],
    )(_sc_body)
    out, _, _ = sc(eidx, tabT)
    return out.reshape(E)
